# baseline (speedup 1.0000x reference)
import functools

import jax
import jax.numpy as jnp
from jax import lax
from jax.experimental import pallas as pl
from jax.experimental.pallas import tpu as pltpu
from jax.experimental.pallas import tpu_sc as plsc

N = 100000
E = 1600000
F = 128
EMB = 50
HID = 16
NCLS = 10
NGRAPH = 64

NC, NS, L = 2, 16, 16
NW = NC * NS
W = 16
NACC = 100352
RPT = NACC // NS
NFLAT = NACC * W // 128

KD = 2000
CD = E // NW
KM = 768
NCK0 = 50
NCK1 = 90
NCHUNK = NCK0 + NCK1
EP = KM * NS * NCHUNK

_f32 = jnp.float32
_i32 = jnp.int32

_SC_PARAMS = pltpu.CompilerParams(needs_layout_passes=False,
                                  use_tc_tiling_on_sc=False)


@functools.cache
def _mesh():
    return plsc.VectorSubcoreMesh(core_axis_name="c", subcore_axis_name="s",
                                  num_cores=NC, num_subcores=NS)


def _deg_body(dst_hbm, ew_hbm, degp_hbm, dstb, wb, zb, acc):
    cid = lax.axis_index("c")
    sid = lax.axis_index("s")
    wid = sid * NC + cid

    def zfill(j, carry):
        zb[pl.ds(j * L, L)] = jnp.zeros((L,), _f32)
        return carry
    lax.fori_loop(0, RPT // L, zfill, 0)
    pltpu.sync_copy(zb, acc.at[pl.ds(sid * RPT, RPT)])
    plsc.subcore_barrier()

    base = wid * CD

    def chunk(j, carry):
        off = base + j * KD
        pltpu.sync_copy(dst_hbm.at[pl.ds(off, KD)], dstb)
        pltpu.sync_copy(ew_hbm.at[pl.ds(off, KD)], wb)
        pltpu.sync_copy(wb, acc.at[dstb], add=True)
        return carry
    lax.fori_loop(0, CD // KD, chunk, 0)
    plsc.subcore_barrier()
    pltpu.sync_copy(acc.at[pl.ds(sid * RPT, RPT)],
                    degp_hbm.at[cid, pl.ds(sid * RPT, RPT)])


@functools.cache
def _deg_kernel():
    return pl.kernel(
        _deg_body,
        out_type=jax.ShapeDtypeStruct((NC, NACC), _f32),
        mesh=_mesh(),
        compiler_params=_SC_PARAMS,
        scratch_types=[
            pltpu.VMEM((KD,), _i32),
            pltpu.VMEM((KD,), _f32),
            pltpu.VMEM((RPT,), _f32),
            pltpu.VMEM_SHARED((NACC,), _f32),
        ],
    )


def _deg_call(dst, ew):
    return _deg_kernel()(dst, ew)


def _msg_body(src_hbm, dst_hbm, ewp_hbm, g_hbm, out_hbm,
              srcb0, srcb1, dstb0, dstb1, ewb0, ewb1, rows0, rows1,
              semg0, semg1, sems0, sems1, semi0, semi1, acc):
    cid = lax.axis_index("c")
    sid = lax.axis_index("s")
    wid = sid * NC + cid

    srcb = (srcb0, srcb1)
    dstb = (dstb0, dstb1)
    ewb = (ewb0, ewb1)
    rows = (rows0, rows1)
    semg = (semg0, semg1)
    sems = (sems0, sems1)
    semi = (semi0, semi1)

    nck = jnp.where(cid == 0, NCK0, NCK1)
    npair = nck // 2

    def zfill(i, carry):
        rows0[i, :] = jnp.zeros((L,), _f32)
        return carry
    lax.fori_loop(0, KM, zfill, 0)
    r0 = sid * RPT
    for t in range(RPT // KM):
        pltpu.sync_copy(rows0, acc.at[pl.ds(r0 + t * KM, KM), :])
    rem = RPT % KM
    if rem:
        pltpu.sync_copy(rows0.at[pl.ds(0, rem), :],
                        acc.at[pl.ds(r0 + (RPT // KM) * KM, rem), :])
    plsc.subcore_barrier()

    base = jnp.where(cid == 0, sid * (NCK0 * KM),
                     NS * NCK0 * KM + sid * (NCK1 * KM))

    def start_idx(c, b):
        off = base + c * KM
        pltpu.async_copy(src_hbm.at[pl.ds(off, KM)], srcb[b], semi[b])
        pltpu.async_copy(dst_hbm.at[pl.ds(off, KM)], dstb[b], semi[b])
        pltpu.async_copy(ewp_hbm.at[pl.ds(off, KM)], ewb[b], semi[b])

    def wait_idx(c, b):
        off = base + c * KM
        pltpu.make_async_copy(src_hbm.at[pl.ds(off, KM)], srcb[b],
                              semi[b]).wait()
        pltpu.make_async_copy(dst_hbm.at[pl.ds(off, KM)], dstb[b],
                              semi[b]).wait()
        pltpu.make_async_copy(ewp_hbm.at[pl.ds(off, KM)], ewb[b],
                              semi[b]).wait()

    def start_gather(b):
        pltpu.async_copy(g_hbm.at[srcb[b]], rows[b], semg[b])

    def wait_gather(b):
        pltpu.make_async_copy(g_hbm.at[srcb[b]], rows[b], semg[b]).wait()

    def start_scat(b):
        pltpu.async_copy(rows[b], acc.at[dstb[b]], sems[b], add=True)

    def wait_scat(b):
        pltpu.make_async_copy(rows[b], acc.at[dstb[b]], sems[b]).wait()

    def scale(b):
        def grp(g, carry):
            nv = ewb[b][pl.ds(g * L, L)]
            for j2 in range(L):
                r = g * L + j2
                rows[b][r, :] = rows[b][r, :] * nv[j2]
            return carry
        lax.fori_loop(0, KM // L, grp, 0)

    def sub_body(c, b):
        @pl.when(c >= 2)
        def _():
            wait_scat(b)
        wait_idx(c, b)
        start_gather(b)

        @pl.when(c >= 1)
        def _():
            wait_gather(b ^ 1)
            scale(b ^ 1)

        @pl.when(c + 1 < nck)
        def _():
            start_idx(c + 1, b ^ 1)

        @pl.when(c >= 1)
        def _():
            start_scat(b ^ 1)

    start_idx(0, 0)

    def pair(p, carry):
        sub_body(2 * p, 0)
        sub_body(2 * p + 1, 1)
        return carry
    lax.fori_loop(0, npair, pair, 0)
    wait_gather(1)
    scale(1)
    wait_scat(0)
    start_scat(1)
    wait_scat(1)

    plsc.subcore_barrier()
    pltpu.sync_copy(acc.at[pl.ds(r0, RPT), :],
                    out_hbm.at[cid, pl.ds(r0, RPT), :])


@functools.cache
def _msg_kernel():
    return pl.kernel(
        _msg_body,
        out_type=jax.ShapeDtypeStruct((NC, NACC, W), _f32),
        mesh=_mesh(),
        compiler_params=_SC_PARAMS,
        scratch_types=[
            pltpu.VMEM((KM,), _i32),
            pltpu.VMEM((KM,), _i32),
            pltpu.VMEM((KM,), _i32),
            pltpu.VMEM((KM,), _i32),
            pltpu.VMEM((KM,), _f32),
            pltpu.VMEM((KM,), _f32),
            pltpu.VMEM((KM, W), _f32),
            pltpu.VMEM((KM, W), _f32),
            pltpu.SemaphoreType.DMA,
            pltpu.SemaphoreType.DMA,
            pltpu.SemaphoreType.DMA,
            pltpu.SemaphoreType.DMA,
            pltpu.SemaphoreType.DMA,
            pltpu.SemaphoreType.DMA,
            pltpu.VMEM_SHARED((NACC, W), _f32),
        ],
    )


def _msg_call(srcp, dstp, ewp, gs):
    return _msg_kernel()(srcp, dstp, ewp, gs)


_DB = 2048


def _dinvrep_body(degp_ref, out_ref):
    deg = degp_ref[0, 0] + degp_ref[1, 0] + 1.0
    dinv = jnp.where(deg > 0, lax.rsqrt(jnp.maximum(deg, 1e-12)), 0.0)
    ident = (lax.broadcasted_iota(_i32, (128, 128), 0)
             == lax.broadcasted_iota(_i32, (128, 128), 1)).astype(_f32)
    t = lax.dot_general(ident, dinv, (((1,), (1,)), ((), ())),
                        preferred_element_type=_f32)
    pieces = [jnp.broadcast_to(t[:, a:a + 1], (128, W))
              for a in range(_DB // 128)]
    out_ref[...] = jnp.concatenate(pieces, axis=0)


def _dinvrep_call(degp):
    return pl.pallas_call(
        _dinvrep_body,
        grid=(NACC // _DB,),
        in_specs=[pl.BlockSpec((NC, 1, _DB // 128, 128),
                               lambda i: (0, i, 0, 0))],
        out_specs=pl.BlockSpec((_DB, W), lambda i: (i, 0)),
        out_shape=jax.ShapeDtypeStruct((NACC, W), _f32),
    )(degp.reshape(NC, NACC // _DB, _DB // 128, 128))


_RB = 2000


def _embed_body(x_ref, emb_ref, w1_ref, drep_ref, g_ref):
    xb = x_ref[...]
    t1 = jnp.dot(emb_ref[...], w1_ref[...], preferred_element_type=_f32)
    m = jnp.max(xb, axis=1, keepdims=True)
    iota = lax.broadcasted_iota(_i32, xb.shape, 1)
    cand = jnp.where(xb == m, iota, F)
    am = jnp.min(cand, axis=1, keepdims=True)
    oh = (iota == am).astype(_f32)
    g_ref[...] = jnp.dot(oh, t1, preferred_element_type=_f32) * drep_ref[...]


def _embed_call(x, emb, w1, drep):
    embp = jnp.pad(emb, ((0, 0), (0, 64 - EMB)))
    w1p = jnp.pad(w1, ((0, 64 - EMB), (0, 0)))
    return pl.pallas_call(
        _embed_body,
        grid=(N // _RB,),
        in_specs=[
            pl.BlockSpec((_RB, F), lambda i: (i, 0)),
            pl.BlockSpec((F, 64), lambda i: (0, 0)),
            pl.BlockSpec((64, W), lambda i: (0, 0)),
            pl.BlockSpec((_RB, W), lambda i: (i, 0)),
        ],
        out_specs=pl.BlockSpec((_RB, W), lambda i: (i, 0)),
        out_shape=jax.ShapeDtypeStruct((N, W), _f32),
    )(x, embp, w1p, drep)


_CB = 256


def _combine_body(p_ref, dflat_ref, brep_ref, w_ref, g_ref):
    dflat = dflat_ref[...]
    h = jnp.maximum(dflat * (p_ref[0] + p_ref[1]) + brep_ref[...], 0.0)
    wt = jnp.tile(w_ref[...], (8, 8))
    ri = lax.broadcasted_iota(_i32, (128, 128), 0) // W
    ci = lax.broadcasted_iota(_i32, (128, 128), 1) // W
    wb = jnp.where(ri == ci, wt, 0.0)
    g_ref[...] = jnp.dot(h, wb, preferred_element_type=_f32) * dflat


def _combine_call(p, drep, b, w):
    brep = jnp.tile(b, 8).reshape(1, 128)
    return pl.pallas_call(
        _combine_body,
        grid=(NFLAT // _CB,),
        in_specs=[
            pl.BlockSpec((NC, _CB, 128), lambda i: (0, i, 0)),
            pl.BlockSpec((_CB, 128), lambda i: (i, 0)),
            pl.BlockSpec((1, 128), lambda i: (0, 0)),
            pl.BlockSpec((W, W), lambda i: (0, 0)),
        ],
        out_specs=pl.BlockSpec((_CB, 128), lambda i: (i, 0)),
        out_shape=jax.ShapeDtypeStruct((NFLAT, 128), _f32),
    )(p.reshape(NC, NFLAT, 128), drep.reshape(NFLAT, 128), brep, w)


def _pool_body(p_ref, drep_ref, b3_ref, batch_ref, out_ref, acc_ref):
    i = pl.program_id(0)

    @pl.when(i == 0)
    def _():
        acc_ref[...] = jnp.zeros_like(acc_ref)

    h3 = drep_ref[...] * (p_ref[0] + p_ref[1]) + b3_ref[...]
    bb = batch_ref[0, 0, :]
    oh = (bb[:, None] == lax.broadcasted_iota(_i32, (_RB, NGRAPH), 1))
    oh = oh.astype(_f32)
    h3a = jnp.concatenate([h3, jnp.ones_like(h3)], axis=1)
    acc_ref[...] += lax.dot_general(oh, h3a, (((0,), (0,)), ((), ())),
                                    preferred_element_type=_f32)

    @pl.when(i == pl.num_programs(0) - 1)
    def _():
        accv = acc_ref[...]
        cnt = jnp.maximum(accv[:, W:W + 1], 1.0)
        pooled = accv[:, :NCLS] / cnt
        mx = jnp.max(pooled, axis=1, keepdims=True)
        e = jnp.exp(pooled - mx)
        out_ref[...] = e / jnp.sum(e, axis=1, keepdims=True)


def _pool_call(p, drep, b3p, batch):
    return pl.pallas_call(
        _pool_body,
        grid=(N // _RB,),
        in_specs=[
            pl.BlockSpec((NC, _RB, W), lambda i: (0, i, 0)),
            pl.BlockSpec((_RB, W), lambda i: (i, 0)),
            pl.BlockSpec((1, W), lambda i: (0, 0)),
            pl.BlockSpec((1, 1, _RB), lambda i: (i, 0, 0)),
        ],
        out_specs=pl.BlockSpec((NGRAPH, NCLS), lambda i: (0, 0)),
        out_shape=jax.ShapeDtypeStruct((NGRAPH, NCLS), _f32),
        scratch_shapes=[pltpu.VMEM((NGRAPH, 2 * W), _f32)],
    )(p, drep, b3p.reshape(1, W), batch.reshape(N // _RB, 1, _RB))


def kernel(x, edge_index, edge_attr, batch, emb, W1, b1, W2, b2, W3, b3):
    src = edge_index[0]
    dst = edge_index[1]

    degp = _deg_call(dst, edge_attr)
    drep = _dinvrep_call(degp)
    gs1 = _embed_call(x, emb, W1, drep)

    loop_idx = jnp.arange(N, dtype=_i32)
    pad = EP - E - N
    srcp = jnp.concatenate([src, loop_idx, jnp.zeros((pad,), _i32)])
    dstp = jnp.concatenate([dst, loop_idx, jnp.full((pad,), N, _i32)])
    ewp = jnp.concatenate([edge_attr, jnp.ones((N,), _f32),
                           jnp.zeros((pad,), _f32)])

    p1 = _msg_call(srcp, dstp, ewp, gs1)
    gs2 = _combine_call(p1, drep, b1, W2).reshape(NACC, W)
    p2 = _msg_call(srcp, dstp, ewp, gs2)
    w3p = jnp.pad(W3, ((0, 0), (0, W - NCLS)))
    b3p = jnp.pad(b3, (0, W - NCLS))
    gs3 = _combine_call(p2, drep, b2, w3p).reshape(NACC, W)
    p3 = _msg_call(srcp, dstp, ewp, gs3)

    return _pool_call(p3, drep, b3p, batch)


# speedup vs baseline: 1.1278x; 1.1278x over previous
import functools

import jax
import jax.numpy as jnp
from jax import lax
from jax.experimental import pallas as pl
from jax.experimental.pallas import tpu as pltpu
from jax.experimental.pallas import tpu_sc as plsc

N = 100000
E = 1600000
F = 128
EMB = 50
HID = 16
NCLS = 10
NGRAPH = 64

NC, NS, L = 2, 16, 16
NW = NC * NS
W = 16
NACC = 100352
RPT = NACC // NS
NFLAT = NACC * W // 128

KD = 2000
CD = E // NW
KM = 768
NCK0 = 90
NCK1 = 50
NCHUNK = NCK0 + NCK1
EP = KM * NS * NCHUNK

_f32 = jnp.float32
_i32 = jnp.int32

_SC_PARAMS = pltpu.CompilerParams(needs_layout_passes=False,
                                  use_tc_tiling_on_sc=False)


@functools.cache
def _mesh():
    return plsc.VectorSubcoreMesh(core_axis_name="c", subcore_axis_name="s",
                                  num_cores=NC, num_subcores=NS)


def _deg_body(dst_hbm, ew_hbm, degp_hbm, dstb, wb, zb, acc):
    cid = lax.axis_index("c")
    sid = lax.axis_index("s")
    wid = sid * NC + cid

    def zfill(j, carry):
        zb[pl.ds(j * L, L)] = jnp.zeros((L,), _f32)
        return carry
    lax.fori_loop(0, RPT // L, zfill, 0)
    pltpu.sync_copy(zb, acc.at[pl.ds(sid * RPT, RPT)])
    plsc.subcore_barrier()

    base = wid * CD

    def chunk(j, carry):
        off = base + j * KD
        pltpu.sync_copy(dst_hbm.at[pl.ds(off, KD)], dstb)
        pltpu.sync_copy(ew_hbm.at[pl.ds(off, KD)], wb)
        pltpu.sync_copy(wb, acc.at[dstb], add=True)
        return carry
    lax.fori_loop(0, CD // KD, chunk, 0)
    plsc.subcore_barrier()
    pltpu.sync_copy(acc.at[pl.ds(sid * RPT, RPT)],
                    degp_hbm.at[cid, pl.ds(sid * RPT, RPT)])


@functools.cache
def _deg_kernel():
    return pl.kernel(
        _deg_body,
        out_type=jax.ShapeDtypeStruct((NC, NACC), _f32),
        mesh=_mesh(),
        compiler_params=_SC_PARAMS,
        scratch_types=[
            pltpu.VMEM((KD,), _i32),
            pltpu.VMEM((KD,), _f32),
            pltpu.VMEM((RPT,), _f32),
            pltpu.VMEM_SHARED((NACC,), _f32),
        ],
    )


def _deg_call(dst, ew):
    return _deg_kernel()(dst, ew)


def _msg_body(src_hbm, dst_hbm, ewp_hbm, g_hbm, out_hbm,
              srcb0, srcb1, dstb0, dstb1, ewb0, ewb1, rows0, rows1,
              semg0, semg1, sems0, sems1, semi0, semi1, acc):
    cid = lax.axis_index("c")
    sid = lax.axis_index("s")
    wid = sid * NC + cid

    srcb = (srcb0, srcb1)
    dstb = (dstb0, dstb1)
    ewb = (ewb0, ewb1)
    rows = (rows0, rows1)
    semg = (semg0, semg1)
    sems = (sems0, sems1)
    semi = (semi0, semi1)

    nck = jnp.where(cid == 0, NCK0, NCK1)
    npair = nck // 2

    def zfill(i, carry):
        rows0[i, :] = jnp.zeros((L,), _f32)
        return carry
    lax.fori_loop(0, KM, zfill, 0)
    r0 = sid * RPT
    for t in range(RPT // KM):
        pltpu.sync_copy(rows0, acc.at[pl.ds(r0 + t * KM, KM), :])
    rem = RPT % KM
    if rem:
        pltpu.sync_copy(rows0.at[pl.ds(0, rem), :],
                        acc.at[pl.ds(r0 + (RPT // KM) * KM, rem), :])
    plsc.subcore_barrier()

    base = jnp.where(cid == 0, sid * (NCK0 * KM),
                     NS * NCK0 * KM + sid * (NCK1 * KM))

    def start_idx(c, b):
        off = base + c * KM
        pltpu.async_copy(src_hbm.at[pl.ds(off, KM)], srcb[b], semi[b])
        pltpu.async_copy(dst_hbm.at[pl.ds(off, KM)], dstb[b], semi[b])
        pltpu.async_copy(ewp_hbm.at[pl.ds(off, KM)], ewb[b], semi[b])

    def wait_idx(c, b):
        off = base + c * KM
        pltpu.make_async_copy(src_hbm.at[pl.ds(off, KM)], srcb[b],
                              semi[b]).wait()
        pltpu.make_async_copy(dst_hbm.at[pl.ds(off, KM)], dstb[b],
                              semi[b]).wait()
        pltpu.make_async_copy(ewp_hbm.at[pl.ds(off, KM)], ewb[b],
                              semi[b]).wait()

    def start_gather(b):
        pltpu.async_copy(g_hbm.at[srcb[b]], rows[b], semg[b])

    def wait_gather(b):
        pltpu.make_async_copy(g_hbm.at[srcb[b]], rows[b], semg[b]).wait()

    def start_scat(b):
        pltpu.async_copy(rows[b], acc.at[dstb[b]], sems[b], add=True)

    def wait_scat(b):
        pltpu.make_async_copy(rows[b], acc.at[dstb[b]], sems[b]).wait()

    def scale(b):
        def grp(g, carry):
            nv = ewb[b][pl.ds(g * L, L)]
            for j2 in range(L):
                r = g * L + j2
                rows[b][r, :] = rows[b][r, :] * nv[j2]
            return carry
        lax.fori_loop(0, KM // L, grp, 0)

    def sub_body(c, b):
        @pl.when(c >= 2)
        def _():
            wait_scat(b)
        wait_idx(c, b)
        start_gather(b)

        @pl.when(c >= 1)
        def _():
            wait_gather(b ^ 1)
            scale(b ^ 1)

        @pl.when(c + 1 < nck)
        def _():
            start_idx(c + 1, b ^ 1)

        @pl.when(c >= 1)
        def _():
            start_scat(b ^ 1)

    start_idx(0, 0)

    def pair(p, carry):
        sub_body(2 * p, 0)
        sub_body(2 * p + 1, 1)
        return carry
    lax.fori_loop(0, npair, pair, 0)
    wait_gather(1)
    scale(1)
    wait_scat(0)
    start_scat(1)
    wait_scat(1)

    plsc.subcore_barrier()
    pltpu.sync_copy(acc.at[pl.ds(r0, RPT), :],
                    out_hbm.at[cid, pl.ds(r0, RPT), :])


@functools.cache
def _msg_kernel():
    return pl.kernel(
        _msg_body,
        out_type=jax.ShapeDtypeStruct((NC, NACC, W), _f32),
        mesh=_mesh(),
        compiler_params=_SC_PARAMS,
        scratch_types=[
            pltpu.VMEM((KM,), _i32),
            pltpu.VMEM((KM,), _i32),
            pltpu.VMEM((KM,), _i32),
            pltpu.VMEM((KM,), _i32),
            pltpu.VMEM((KM,), _f32),
            pltpu.VMEM((KM,), _f32),
            pltpu.VMEM((KM, W), _f32),
            pltpu.VMEM((KM, W), _f32),
            pltpu.SemaphoreType.DMA,
            pltpu.SemaphoreType.DMA,
            pltpu.SemaphoreType.DMA,
            pltpu.SemaphoreType.DMA,
            pltpu.SemaphoreType.DMA,
            pltpu.SemaphoreType.DMA,
            pltpu.VMEM_SHARED((NACC, W), _f32),
        ],
    )


def _msg_call(srcp, dstp, ewp, gs):
    return _msg_kernel()(srcp, dstp, ewp, gs)


_DB = 2048


def _dinvrep_body(degp_ref, out_ref):
    deg = degp_ref[0, 0] + degp_ref[1, 0] + 1.0
    dinv = jnp.where(deg > 0, lax.rsqrt(jnp.maximum(deg, 1e-12)), 0.0)
    ident = (lax.broadcasted_iota(_i32, (128, 128), 0)
             == lax.broadcasted_iota(_i32, (128, 128), 1)).astype(_f32)
    t = lax.dot_general(ident, dinv, (((1,), (1,)), ((), ())),
                        preferred_element_type=_f32)
    pieces = [jnp.broadcast_to(t[:, a:a + 1], (128, W))
              for a in range(_DB // 128)]
    out_ref[...] = jnp.concatenate(pieces, axis=0)


def _dinvrep_call(degp):
    return pl.pallas_call(
        _dinvrep_body,
        grid=(NACC // _DB,),
        in_specs=[pl.BlockSpec((NC, 1, _DB // 128, 128),
                               lambda i: (0, i, 0, 0))],
        out_specs=pl.BlockSpec((_DB, W), lambda i: (i, 0)),
        out_shape=jax.ShapeDtypeStruct((NACC, W), _f32),
    )(degp.reshape(NC, NACC // _DB, _DB // 128, 128))


_RB = 2000


def _embed_body(x_ref, emb_ref, w1_ref, drep_ref, g_ref):
    xb = x_ref[...]
    t1 = jnp.dot(emb_ref[...], w1_ref[...], preferred_element_type=_f32)
    m = jnp.max(xb, axis=1, keepdims=True)
    iota = lax.broadcasted_iota(_i32, xb.shape, 1)
    cand = jnp.where(xb == m, iota, F)
    am = jnp.min(cand, axis=1, keepdims=True)
    oh = (iota == am).astype(_f32)
    g_ref[...] = jnp.dot(oh, t1, preferred_element_type=_f32) * drep_ref[...]


def _embed_call(x, emb, w1, drep):
    embp = jnp.pad(emb, ((0, 0), (0, 64 - EMB)))
    w1p = jnp.pad(w1, ((0, 64 - EMB), (0, 0)))
    return pl.pallas_call(
        _embed_body,
        grid=(N // _RB,),
        in_specs=[
            pl.BlockSpec((_RB, F), lambda i: (i, 0)),
            pl.BlockSpec((F, 64), lambda i: (0, 0)),
            pl.BlockSpec((64, W), lambda i: (0, 0)),
            pl.BlockSpec((_RB, W), lambda i: (i, 0)),
        ],
        out_specs=pl.BlockSpec((_RB, W), lambda i: (i, 0)),
        out_shape=jax.ShapeDtypeStruct((N, W), _f32),
    )(x, embp, w1p, drep)


_CB = 256


def _combine_body(p_ref, dflat_ref, brep_ref, w_ref, g_ref):
    dflat = dflat_ref[...]
    h = jnp.maximum(dflat * (p_ref[0] + p_ref[1]) + brep_ref[...], 0.0)
    wt = jnp.tile(w_ref[...], (8, 8))
    ri = lax.broadcasted_iota(_i32, (128, 128), 0) // W
    ci = lax.broadcasted_iota(_i32, (128, 128), 1) // W
    wb = jnp.where(ri == ci, wt, 0.0)
    g_ref[...] = jnp.dot(h, wb, preferred_element_type=_f32) * dflat


def _combine_call(p, drep, b, w):
    brep = jnp.tile(b, 8).reshape(1, 128)
    return pl.pallas_call(
        _combine_body,
        grid=(NFLAT // _CB,),
        in_specs=[
            pl.BlockSpec((NC, _CB, 128), lambda i: (0, i, 0)),
            pl.BlockSpec((_CB, 128), lambda i: (i, 0)),
            pl.BlockSpec((1, 128), lambda i: (0, 0)),
            pl.BlockSpec((W, W), lambda i: (0, 0)),
        ],
        out_specs=pl.BlockSpec((_CB, 128), lambda i: (i, 0)),
        out_shape=jax.ShapeDtypeStruct((NFLAT, 128), _f32),
    )(p.reshape(NC, NFLAT, 128), drep.reshape(NFLAT, 128), brep, w)


def _pool_body(p_ref, drep_ref, b3_ref, batch_ref, out_ref, acc_ref):
    i = pl.program_id(0)

    @pl.when(i == 0)
    def _():
        acc_ref[...] = jnp.zeros_like(acc_ref)

    h3 = drep_ref[...] * (p_ref[0] + p_ref[1]) + b3_ref[...]
    bb = batch_ref[0, 0, :]
    oh = (bb[:, None] == lax.broadcasted_iota(_i32, (_RB, NGRAPH), 1))
    oh = oh.astype(_f32)
    h3a = jnp.concatenate([h3, jnp.ones_like(h3)], axis=1)
    acc_ref[...] += lax.dot_general(oh, h3a, (((0,), (0,)), ((), ())),
                                    preferred_element_type=_f32)

    @pl.when(i == pl.num_programs(0) - 1)
    def _():
        accv = acc_ref[...]
        cnt = jnp.maximum(accv[:, W:W + 1], 1.0)
        pooled = accv[:, :NCLS] / cnt
        mx = jnp.max(pooled, axis=1, keepdims=True)
        e = jnp.exp(pooled - mx)
        out_ref[...] = e / jnp.sum(e, axis=1, keepdims=True)


def _pool_call(p, drep, b3p, batch):
    return pl.pallas_call(
        _pool_body,
        grid=(N // _RB,),
        in_specs=[
            pl.BlockSpec((NC, _RB, W), lambda i: (0, i, 0)),
            pl.BlockSpec((_RB, W), lambda i: (i, 0)),
            pl.BlockSpec((1, W), lambda i: (0, 0)),
            pl.BlockSpec((1, 1, _RB), lambda i: (i, 0, 0)),
        ],
        out_specs=pl.BlockSpec((NGRAPH, NCLS), lambda i: (0, 0)),
        out_shape=jax.ShapeDtypeStruct((NGRAPH, NCLS), _f32),
        scratch_shapes=[pltpu.VMEM((NGRAPH, 2 * W), _f32)],
    )(p, drep, b3p.reshape(1, W), batch.reshape(N // _RB, 1, _RB))


def kernel(x, edge_index, edge_attr, batch, emb, W1, b1, W2, b2, W3, b3):
    src = edge_index[0]
    dst = edge_index[1]

    degp = _deg_call(dst, edge_attr)
    drep = _dinvrep_call(degp)
    gs1 = _embed_call(x, emb, W1, drep)

    loop_idx = jnp.arange(N, dtype=_i32)
    pad = EP - E - N
    srcp = jnp.concatenate([src, loop_idx, jnp.zeros((pad,), _i32)])
    dstp = jnp.concatenate([dst, loop_idx, jnp.full((pad,), N, _i32)])
    ewp = jnp.concatenate([edge_attr, jnp.ones((N,), _f32),
                           jnp.zeros((pad,), _f32)])

    p1 = _msg_call(srcp, dstp, ewp, gs1)
    gs2 = _combine_call(p1, drep, b1, W2).reshape(NACC, W)
    p2 = _msg_call(srcp, dstp, ewp, gs2)
    w3p = jnp.pad(W3, ((0, 0), (0, W - NCLS)))
    b3p = jnp.pad(b3, (0, W - NCLS))
    gs3 = _combine_call(p2, drep, b2, w3p).reshape(NACC, W)
    p3 = _msg_call(srcp, dstp, ewp, gs3)

    return _pool_call(p3, drep, b3p, batch)


# speedup vs baseline: 1.2685x; 1.1247x over previous
import functools

import jax
import jax.numpy as jnp
from jax import lax
from jax.experimental import pallas as pl
from jax.experimental.pallas import tpu as pltpu
from jax.experimental.pallas import tpu_sc as plsc

N = 100000
E = 1600000
F = 128
EMB = 50
HID = 16
NCLS = 10
NGRAPH = 64

NC, NS, L = 2, 16, 16
NW = NC * NS
W = 16
NACC = 100352
RPT = NACC // NS
NFLAT = NACC * W // 128

KD = 2000
CD = E // NW
KM = 768
NCK0 = 90
NCK1 = 50
NCHUNK = NCK0 + NCK1
EP = KM * NS * NCHUNK

_f32 = jnp.float32
_i32 = jnp.int32

_SC_PARAMS = pltpu.CompilerParams(needs_layout_passes=False,
                                  use_tc_tiling_on_sc=False)


@functools.cache
def _mesh():
    return plsc.VectorSubcoreMesh(core_axis_name="c", subcore_axis_name="s",
                                  num_cores=NC, num_subcores=NS)


SELF0 = E
SELFN = NACC // NW
PAD0 = E + NACC
PADN = (EP - PAD0) // NW


def _deg_body(ei_hbm, ew_hbm, degp_hbm, srcp_hbm, dstp_hbm, ewp_hbm,
              srcb, dstb, wb, zb, ib, sfb, acc):
    cid = lax.axis_index("c")
    sid = lax.axis_index("s")
    wid = sid * NC + cid

    def zfill(j, carry):
        zb[pl.ds(j * L, L)] = jnp.zeros((L,), _f32)
        return carry
    lax.fori_loop(0, RPT // L, zfill, 0)
    pltpu.sync_copy(zb, acc.at[pl.ds(sid * RPT, RPT)])

    nb = wid * SELFN
    iota16 = lax.iota(_i32, L)

    def sfill(i, carry):
        v = nb + i * L + iota16
        valid = v < N
        ib[pl.ds(i * L, L)] = jnp.where(valid, v, 0)
        sfb[pl.ds(i * L, L)] = jnp.where(valid, 1.0, 0.0)
        return carry
    lax.fori_loop(0, SELFN // L, sfill, 0)
    pltpu.sync_copy(ib, srcp_hbm.at[pl.ds(SELF0 + nb, SELFN)])
    pltpu.sync_copy(ib, dstp_hbm.at[pl.ds(SELF0 + nb, SELFN)])
    pltpu.sync_copy(sfb, ewp_hbm.at[pl.ds(SELF0 + nb, SELFN)])

    def zfill2(i, carry):
        ib[pl.ds(i * L, L)] = jnp.zeros((L,), _i32)
        sfb[pl.ds(i * L, L)] = jnp.zeros((L,), _f32)
        return carry
    lax.fori_loop(0, PADN // L, zfill2, 0)
    pb = PAD0 + wid * PADN
    pltpu.sync_copy(ib.at[pl.ds(0, PADN)], srcp_hbm.at[pl.ds(pb, PADN)])
    pltpu.sync_copy(ib.at[pl.ds(0, PADN)], dstp_hbm.at[pl.ds(pb, PADN)])
    pltpu.sync_copy(sfb.at[pl.ds(0, PADN)], ewp_hbm.at[pl.ds(pb, PADN)])
    plsc.subcore_barrier()

    base = wid * CD

    def chunk(j, carry):
        off = base + j * KD
        pltpu.sync_copy(ei_hbm.at[0, pl.ds(off, KD)], srcb)
        pltpu.sync_copy(ei_hbm.at[1, pl.ds(off, KD)], dstb)
        pltpu.sync_copy(ew_hbm.at[pl.ds(off, KD)], wb)
        pltpu.sync_copy(srcb, srcp_hbm.at[pl.ds(off, KD)])
        pltpu.sync_copy(dstb, dstp_hbm.at[pl.ds(off, KD)])
        pltpu.sync_copy(wb, ewp_hbm.at[pl.ds(off, KD)])
        pltpu.sync_copy(wb, acc.at[dstb], add=True)
        return carry
    lax.fori_loop(0, CD // KD, chunk, 0)
    plsc.subcore_barrier()
    pltpu.sync_copy(acc.at[pl.ds(sid * RPT, RPT)],
                    degp_hbm.at[cid, pl.ds(sid * RPT, RPT)])


@functools.cache
def _deg_kernel():
    return pl.kernel(
        _deg_body,
        out_type=(jax.ShapeDtypeStruct((NC, NACC), _f32),
                  jax.ShapeDtypeStruct((EP,), _i32),
                  jax.ShapeDtypeStruct((EP,), _i32),
                  jax.ShapeDtypeStruct((EP,), _f32)),
        mesh=_mesh(),
        compiler_params=_SC_PARAMS,
        scratch_types=[
            pltpu.VMEM((KD,), _i32),
            pltpu.VMEM((KD,), _i32),
            pltpu.VMEM((KD,), _f32),
            pltpu.VMEM((RPT,), _f32),
            pltpu.VMEM((SELFN,), _i32),
            pltpu.VMEM((SELFN,), _f32),
            pltpu.VMEM_SHARED((NACC,), _f32),
        ],
    )


def _deg_call(edge_index, edge_attr):
    return _deg_kernel()(edge_index, edge_attr)


def _msg_body(src_hbm, dst_hbm, ewp_hbm, g_hbm, out_hbm,
              srcb0, srcb1, dstb0, dstb1, ewb0, ewb1, rows0, rows1,
              semg0, semg1, sems0, sems1, semi0, semi1, acc):
    cid = lax.axis_index("c")
    sid = lax.axis_index("s")
    wid = sid * NC + cid

    srcb = (srcb0, srcb1)
    dstb = (dstb0, dstb1)
    ewb = (ewb0, ewb1)
    rows = (rows0, rows1)
    semg = (semg0, semg1)
    sems = (sems0, sems1)
    semi = (semi0, semi1)

    nck = jnp.where(cid == 0, NCK0, NCK1)
    npair = nck // 2

    def zfill(i, carry):
        rows0[i, :] = jnp.zeros((L,), _f32)
        return carry
    lax.fori_loop(0, KM, zfill, 0)
    r0 = sid * RPT
    for t in range(RPT // KM):
        pltpu.sync_copy(rows0, acc.at[pl.ds(r0 + t * KM, KM), :])
    rem = RPT % KM
    if rem:
        pltpu.sync_copy(rows0.at[pl.ds(0, rem), :],
                        acc.at[pl.ds(r0 + (RPT // KM) * KM, rem), :])
    plsc.subcore_barrier()

    base = jnp.where(cid == 0, sid * (NCK0 * KM),
                     NS * NCK0 * KM + sid * (NCK1 * KM))

    def start_idx(c, b):
        off = base + c * KM
        pltpu.async_copy(src_hbm.at[pl.ds(off, KM)], srcb[b], semi[b])
        pltpu.async_copy(dst_hbm.at[pl.ds(off, KM)], dstb[b], semi[b])
        pltpu.async_copy(ewp_hbm.at[pl.ds(off, KM)], ewb[b], semi[b])

    def wait_idx(c, b):
        off = base + c * KM
        pltpu.make_async_copy(src_hbm.at[pl.ds(off, KM)], srcb[b],
                              semi[b]).wait()
        pltpu.make_async_copy(dst_hbm.at[pl.ds(off, KM)], dstb[b],
                              semi[b]).wait()
        pltpu.make_async_copy(ewp_hbm.at[pl.ds(off, KM)], ewb[b],
                              semi[b]).wait()

    def start_gather(b):
        pltpu.async_copy(g_hbm.at[srcb[b]], rows[b], semg[b])

    def wait_gather(b):
        pltpu.make_async_copy(g_hbm.at[srcb[b]], rows[b], semg[b]).wait()

    def start_scat(b):
        pltpu.async_copy(rows[b], acc.at[dstb[b]], sems[b], add=True)

    def wait_scat(b):
        pltpu.make_async_copy(rows[b], acc.at[dstb[b]], sems[b]).wait()

    def scale(b):
        def grp(g, carry):
            nv = ewb[b][pl.ds(g * L, L)]
            for j2 in range(L):
                r = g * L + j2
                rows[b][r, :] = rows[b][r, :] * nv[j2]
            return carry
        lax.fori_loop(0, KM // L, grp, 0)

    def sub_body(c, b):
        @pl.when(c >= 2)
        def _():
            wait_scat(b)
        wait_idx(c, b)
        start_gather(b)

        @pl.when(c >= 1)
        def _():
            wait_gather(b ^ 1)
            scale(b ^ 1)

        @pl.when(c + 1 < nck)
        def _():
            start_idx(c + 1, b ^ 1)

        @pl.when(c >= 1)
        def _():
            start_scat(b ^ 1)

    start_idx(0, 0)

    def pair(p, carry):
        sub_body(2 * p, 0)
        sub_body(2 * p + 1, 1)
        return carry
    lax.fori_loop(0, npair, pair, 0)
    wait_gather(1)
    scale(1)
    wait_scat(0)
    start_scat(1)
    wait_scat(1)

    plsc.subcore_barrier()
    pltpu.sync_copy(acc.at[pl.ds(r0, RPT), :],
                    out_hbm.at[cid, pl.ds(r0, RPT), :])


@functools.cache
def _msg_kernel():
    return pl.kernel(
        _msg_body,
        out_type=jax.ShapeDtypeStruct((NC, NACC, W), _f32),
        mesh=_mesh(),
        compiler_params=_SC_PARAMS,
        scratch_types=[
            pltpu.VMEM((KM,), _i32),
            pltpu.VMEM((KM,), _i32),
            pltpu.VMEM((KM,), _i32),
            pltpu.VMEM((KM,), _i32),
            pltpu.VMEM((KM,), _f32),
            pltpu.VMEM((KM,), _f32),
            pltpu.VMEM((KM, W), _f32),
            pltpu.VMEM((KM, W), _f32),
            pltpu.SemaphoreType.DMA,
            pltpu.SemaphoreType.DMA,
            pltpu.SemaphoreType.DMA,
            pltpu.SemaphoreType.DMA,
            pltpu.SemaphoreType.DMA,
            pltpu.SemaphoreType.DMA,
            pltpu.VMEM_SHARED((NACC, W), _f32),
        ],
    )


def _msg_call(srcp, dstp, ewp, gs):
    return _msg_kernel()(srcp, dstp, ewp, gs)


_DB = 2048


def _dinvrep_body(degp_ref, out_ref):
    deg = degp_ref[0, 0] + degp_ref[1, 0] + 1.0
    dinv = jnp.where(deg > 0, lax.rsqrt(jnp.maximum(deg, 1e-12)), 0.0)
    ident = (lax.broadcasted_iota(_i32, (128, 128), 0)
             == lax.broadcasted_iota(_i32, (128, 128), 1)).astype(_f32)
    t = lax.dot_general(ident, dinv, (((1,), (1,)), ((), ())),
                        preferred_element_type=_f32)
    pieces = [jnp.broadcast_to(t[:, a:a + 1], (128, W))
              for a in range(_DB // 128)]
    out_ref[...] = jnp.concatenate(pieces, axis=0)


def _dinvrep_call(degp):
    return pl.pallas_call(
        _dinvrep_body,
        grid=(NACC // _DB,),
        in_specs=[pl.BlockSpec((NC, 1, _DB // 128, 128),
                               lambda i: (0, i, 0, 0))],
        out_specs=pl.BlockSpec((_DB, W), lambda i: (i, 0)),
        out_shape=jax.ShapeDtypeStruct((NACC, W), _f32),
    )(degp.reshape(NC, NACC // _DB, _DB // 128, 128))


_RB = 2000


def _embed_body(x_ref, emb_ref, w1_ref, drep_ref, g_ref):
    xb = x_ref[...]
    t1 = jnp.dot(emb_ref[...], w1_ref[...], preferred_element_type=_f32)
    m = jnp.max(xb, axis=1, keepdims=True)
    iota = lax.broadcasted_iota(_i32, xb.shape, 1)
    cand = jnp.where(xb == m, iota, F)
    am = jnp.min(cand, axis=1, keepdims=True)
    oh = (iota == am).astype(_f32)
    g_ref[...] = jnp.dot(oh, t1, preferred_element_type=_f32) * drep_ref[...]


def _embed_call(x, emb, w1, drep):
    embp = jnp.pad(emb, ((0, 0), (0, 64 - EMB)))
    w1p = jnp.pad(w1, ((0, 64 - EMB), (0, 0)))
    return pl.pallas_call(
        _embed_body,
        grid=(N // _RB,),
        in_specs=[
            pl.BlockSpec((_RB, F), lambda i: (i, 0)),
            pl.BlockSpec((F, 64), lambda i: (0, 0)),
            pl.BlockSpec((64, W), lambda i: (0, 0)),
            pl.BlockSpec((_RB, W), lambda i: (i, 0)),
        ],
        out_specs=pl.BlockSpec((_RB, W), lambda i: (i, 0)),
        out_shape=jax.ShapeDtypeStruct((N, W), _f32),
    )(x, embp, w1p, drep)


_CB = 256


def _combine_body(p_ref, dflat_ref, brep_ref, w_ref, g_ref):
    dflat = dflat_ref[...]
    h = jnp.maximum(dflat * (p_ref[0] + p_ref[1]) + brep_ref[...], 0.0)
    wt = jnp.tile(w_ref[...], (8, 8))
    ri = lax.broadcasted_iota(_i32, (128, 128), 0) // W
    ci = lax.broadcasted_iota(_i32, (128, 128), 1) // W
    wb = jnp.where(ri == ci, wt, 0.0)
    g_ref[...] = jnp.dot(h, wb, preferred_element_type=_f32) * dflat


def _combine_call(p, drep, b, w):
    brep = jnp.tile(b, 8).reshape(1, 128)
    return pl.pallas_call(
        _combine_body,
        grid=(NFLAT // _CB,),
        in_specs=[
            pl.BlockSpec((NC, _CB, 128), lambda i: (0, i, 0)),
            pl.BlockSpec((_CB, 128), lambda i: (i, 0)),
            pl.BlockSpec((1, 128), lambda i: (0, 0)),
            pl.BlockSpec((W, W), lambda i: (0, 0)),
        ],
        out_specs=pl.BlockSpec((_CB, 128), lambda i: (i, 0)),
        out_shape=jax.ShapeDtypeStruct((NFLAT, 128), _f32),
    )(p.reshape(NC, NFLAT, 128), drep.reshape(NFLAT, 128), brep, w)


def _pool_body(p_ref, drep_ref, b3_ref, batch_ref, out_ref, acc_ref):
    i = pl.program_id(0)

    @pl.when(i == 0)
    def _():
        acc_ref[...] = jnp.zeros_like(acc_ref)

    h3 = drep_ref[...] * (p_ref[0] + p_ref[1]) + b3_ref[...]
    bb = batch_ref[0, 0, :]
    oh = (bb[:, None] == lax.broadcasted_iota(_i32, (_RB, NGRAPH), 1))
    oh = oh.astype(_f32)
    h3a = jnp.concatenate([h3, jnp.ones_like(h3)], axis=1)
    acc_ref[...] += lax.dot_general(oh, h3a, (((0,), (0,)), ((), ())),
                                    preferred_element_type=_f32)

    @pl.when(i == pl.num_programs(0) - 1)
    def _():
        accv = acc_ref[...]
        cnt = jnp.maximum(accv[:, W:W + 1], 1.0)
        pooled = accv[:, :NCLS] / cnt
        mx = jnp.max(pooled, axis=1, keepdims=True)
        e = jnp.exp(pooled - mx)
        out_ref[...] = e / jnp.sum(e, axis=1, keepdims=True)


def _pool_call(p, drep, b3p, batch):
    return pl.pallas_call(
        _pool_body,
        grid=(N // _RB,),
        in_specs=[
            pl.BlockSpec((NC, _RB, W), lambda i: (0, i, 0)),
            pl.BlockSpec((_RB, W), lambda i: (i, 0)),
            pl.BlockSpec((1, W), lambda i: (0, 0)),
            pl.BlockSpec((1, 1, _RB), lambda i: (i, 0, 0)),
        ],
        out_specs=pl.BlockSpec((NGRAPH, NCLS), lambda i: (0, 0)),
        out_shape=jax.ShapeDtypeStruct((NGRAPH, NCLS), _f32),
        scratch_shapes=[pltpu.VMEM((NGRAPH, 2 * W), _f32)],
    )(p, drep, b3p.reshape(1, W), batch.reshape(N // _RB, 1, _RB))


def kernel(x, edge_index, edge_attr, batch, emb, W1, b1, W2, b2, W3, b3):
    degp, srcp, dstp, ewp = _deg_call(edge_index, edge_attr)
    drep = _dinvrep_call(degp)
    gs1 = _embed_call(x, emb, W1, drep)

    p1 = _msg_call(srcp, dstp, ewp, gs1)
    gs2 = _combine_call(p1, drep, b1, W2).reshape(NACC, W)
    p2 = _msg_call(srcp, dstp, ewp, gs2)
    w3p = jnp.pad(W3, ((0, 0), (0, W - NCLS)))
    b3p = jnp.pad(b3, (0, W - NCLS))
    gs3 = _combine_call(p2, drep, b2, w3p).reshape(NACC, W)
    p3 = _msg_call(srcp, dstp, ewp, gs3)

    return _pool_call(p3, drep, b3p, batch)
